# baseline (speedup 1.0000x reference)
import functools

import jax
import jax.numpy as jnp
from jax import lax
from jax.experimental import pallas as pl
from jax.experimental.pallas import tpu as pltpu
from jax.experimental.pallas import tpu_sc as plsc

NUM_HEROES = 1000000
EMBED_DIM = 32
BATCH = 16384
HIST = 20
TOTAL = BATCH * HIST

_info = plsc.get_sparse_core_info()
NC, NS = _info.num_cores, _info.num_subcores
NW = NC * NS

_mesh = plsc.VectorSubcoreMesh(core_axis_name="c", subcore_axis_name="s")

NBLK = NUM_HEROES // 128
TAIL = NUM_HEROES - NBLK * 128
BLK_PER_W = NBLK // NW
REM_BLK = NBLK - BLK_PER_W * NW


@functools.partial(
    pl.kernel,
    mesh=_mesh,
    out_type=jax.ShapeDtypeStruct((NUM_HEROES // 4, 128), jnp.float32),
    scratch_types=[
        pltpu.VMEM((EMBED_DIM, 128), jnp.float32),
        pltpu.VMEM((EMBED_DIM, 128), jnp.float32),
    ],
    compiler_params=pltpu.CompilerParams(
        use_tc_tiling_on_sc=True, needs_layout_passes=False
    ),
)
def _relayout(tabt_hbm, tail_hbm, lin_hbm, tv, tb):
    wid = lax.axis_index("s") * NC + lax.axis_index("c")

    iota16 = lax.iota(jnp.int32, 16)

    def do_block(j, width):
        pltpu.sync_copy(tabt_hbm.at[:, pl.ds(j * 128, width)], tv.at[:, pl.ds(0, width)])
        for q in range(width // 4):
            for a in range(8):
                rows = iota16 + (16 * (a % 2))
                cols = jnp.full((16,), 4 * q + a // 2, dtype=jnp.int32)
                tb[q, pl.ds(a * 16, 16)] = plsc.load_gather(tv, [rows, cols])
        pltpu.sync_copy(
            tb.at[pl.ds(0, width // 4), :], lin_hbm.at[pl.ds(j * 32, width // 4), :]
        )

    def body(i, carry):
        do_block(wid * BLK_PER_W + i, 128)
        return carry

    lax.fori_loop(0, BLK_PER_W, body, 0)
    for r in range(REM_BLK):
        @pl.when(wid == r)
        def _():
            do_block(NW * BLK_PER_W + r, 128)
    @pl.when(wid == REM_BLK)
    def _():
        pltpu.sync_copy(tail_hbm, lin_hbm.at[pl.ds(NBLK * 32, TAIL // 4), :])


PER_W = TOTAL // NW
CHUNK = 1280
NCHUNK = PER_W // CHUNK
NB = 2


@functools.partial(
    pl.kernel,
    mesh=_mesh,
    out_type=jax.ShapeDtypeStruct((TOTAL, EMBED_DIM), jnp.float32),
    scratch_types=[
        pltpu.VMEM((NB, CHUNK), jnp.int32),
        pltpu.VMEM((NB, CHUNK, EMBED_DIM), jnp.float32),
        [pltpu.SemaphoreType.DMA] * NB,
        [pltpu.SemaphoreType.DMA] * NB,
    ],
    compiler_params=pltpu.CompilerParams(use_tc_tiling_on_sc=False),
)
def _gather(idx_hbm, table_hbm, out_hbm, idx_v, rows_v, gsems, osems):
    wid = lax.axis_index("s") * NC + lax.axis_index("c")
    base = wid * PER_W

    gathers = [None] * NCHUNK
    writes = [None] * NCHUNK
    for g in range(NCHUNK):
        b = g % NB
        off = base + g * CHUNK
        if g >= NB:
            writes[g - NB].wait()
        pltpu.sync_copy(idx_hbm.at[pl.ds(off, CHUNK)], idx_v.at[b])
        gathers[g] = pltpu.async_copy(table_hbm.at[idx_v.at[b]], rows_v.at[b], gsems[b])
        if g >= 1:
            pb = (g - 1) % NB
            gathers[g - 1].wait()
            writes[g - 1] = pltpu.async_copy(
                rows_v.at[pb], out_hbm.at[pl.ds(base + (g - 1) * CHUNK, CHUNK)], osems[pb]
            )
    last = NCHUNK - 1
    gathers[last].wait()
    writes[last] = pltpu.async_copy(
        rows_v.at[last % NB], out_hbm.at[pl.ds(base + last * CHUNK, CHUNK)], osems[last % NB]
    )
    for g in range(max(0, NCHUNK - NB), NCHUNK):
        writes[g].wait()


def kernel(hero_ids, table):
    tail = table[NBLK * 128 :, :].reshape(TAIL // 4, 128)
    lin128 = _relayout(table.T, tail)
    tab_lin = lin128.reshape(NUM_HEROES, EMBED_DIM)
    ids_hm = hero_ids.T.reshape(TOTAL).astype(jnp.int32)
    out_hm = _gather(ids_hm, tab_lin)
    return out_hm.reshape(HIST, BATCH, EMBED_DIM).transpose(1, 0, 2)


# speedup vs baseline: 1.6628x; 1.6628x over previous
import functools

import jax
import jax.numpy as jnp
from jax import lax
from jax.experimental import pallas as pl
from jax.experimental.pallas import tpu as pltpu
from jax.experimental.pallas import tpu_sc as plsc

NUM_HEROES = 1000000
EMBED_DIM = 32
BATCH = 16384
HIST = 20
TOTAL = BATCH * HIST

_info = plsc.get_sparse_core_info()
NC, NS = _info.num_cores, _info.num_subcores
NW = NC * NS

_mesh = plsc.VectorSubcoreMesh(core_axis_name="c", subcore_axis_name="s")

NBLK = NUM_HEROES // 128
TAIL = NUM_HEROES - NBLK * 128
BLK_PER_W = NBLK // NW
REM_BLK = NBLK - BLK_PER_W * NW


@functools.partial(
    pl.kernel,
    mesh=_mesh,
    out_type=jax.ShapeDtypeStruct((NUM_HEROES // 4, 128), jnp.float32),
    scratch_types=[
        [pltpu.VMEM((EMBED_DIM, 128), jnp.float32)] * 2,
        [pltpu.VMEM((EMBED_DIM, 128), jnp.float32)] * 2,
        [pltpu.SemaphoreType.DMA] * 2,
        [pltpu.SemaphoreType.DMA] * 2,
    ],
    compiler_params=pltpu.CompilerParams(
        use_tc_tiling_on_sc=True, needs_layout_passes=False
    ),
)
def _relayout(tabt_hbm, tail_hbm, lin_hbm, tvs, tbs, isems, osems):
    wid = lax.axis_index("s") * NC + lax.axis_index("c")
    first = wid * BLK_PER_W

    iota16 = lax.iota(jnp.int32, 16)
    perms = [(iota16 + d) % 16 for d in range(16)]

    def transpose(tv, tb):
        for s in range(2):
            rows = iota16 + 16 * s
            for h0 in range(0, 128, 16):
                for d in range(16):
                    p = perms[d]
                    vals = plsc.load_gather(tv, [rows, p + h0])
                    plsc.store_scatter(tb, [(p + h0) // 4, (p % 4) * 32 + rows], vals)

    def start_in(j, b):
        return pltpu.async_copy(
            tabt_hbm.at[:, pl.ds(j * 128, 128)], tvs[b], isems[b]
        )

    def start_out(j, b):
        return pltpu.async_copy(
            tbs[b], lin_hbm.at[pl.ds(j * 32, EMBED_DIM)], osems[b]
        )

    start_in(first, 0).wait()

    def body(i, carry):
        b0 = first + 2 * i
        cin1 = start_in(b0 + 1, 1)

        @pl.when(i > 0)
        def _():
            pltpu.make_async_copy(tabt_hbm.at[:, pl.ds(0, 128)], tvs[0], isems[0]).wait()
            pltpu.make_async_copy(tbs[0], lin_hbm.at[pl.ds(0, EMBED_DIM)], osems[0]).wait()

        transpose(tvs[0], tbs[0])
        start_out(b0, 0)
        nxt = jnp.minimum(b0 + 2, first + BLK_PER_W - 2)
        start_in(nxt, 0)
        cin1.wait()

        @pl.when(i > 0)
        def _():
            pltpu.make_async_copy(tbs[1], lin_hbm.at[pl.ds(0, EMBED_DIM)], osems[1]).wait()

        transpose(tvs[1], tbs[1])
        start_out(b0 + 1, 1)
        return carry

    lax.fori_loop(0, BLK_PER_W // 2, body, 0)
    pltpu.make_async_copy(tbs[0], lin_hbm.at[pl.ds(0, EMBED_DIM)], osems[0]).wait()
    pltpu.make_async_copy(tbs[1], lin_hbm.at[pl.ds(0, EMBED_DIM)], osems[1]).wait()
    pltpu.make_async_copy(tabt_hbm.at[:, pl.ds(0, 128)], tvs[0], isems[0]).wait()

    @pl.when(wid < REM_BLK)
    def _():
        j = NW * BLK_PER_W + wid
        pltpu.sync_copy(tabt_hbm.at[:, pl.ds(j * 128, 128)], tvs[0])
        transpose(tvs[0], tbs[0])
        pltpu.sync_copy(tbs[0], lin_hbm.at[pl.ds(j * 32, EMBED_DIM)])
    @pl.when(wid == REM_BLK)
    def _():
        pltpu.sync_copy(tail_hbm, lin_hbm.at[pl.ds(NBLK * 32, TAIL // 4), :])


PER_W = TOTAL // NW
CHUNK = 1280
NCHUNK = PER_W // CHUNK
NB = 2


@functools.partial(
    pl.kernel,
    mesh=_mesh,
    out_type=jax.ShapeDtypeStruct((TOTAL, EMBED_DIM), jnp.float32),
    scratch_types=[
        pltpu.VMEM((NB, CHUNK), jnp.int32),
        pltpu.VMEM((NB, CHUNK, EMBED_DIM), jnp.float32),
        [pltpu.SemaphoreType.DMA] * NB,
        [pltpu.SemaphoreType.DMA] * NB,
    ],
    compiler_params=pltpu.CompilerParams(use_tc_tiling_on_sc=False),
)
def _gather(idx_hbm, table_hbm, out_hbm, idx_v, rows_v, gsems, osems):
    wid = lax.axis_index("s") * NC + lax.axis_index("c")
    base = wid * PER_W

    gathers = [None] * NCHUNK
    writes = [None] * NCHUNK
    for g in range(NCHUNK):
        b = g % NB
        off = base + g * CHUNK
        if g >= NB:
            writes[g - NB].wait()
        pltpu.sync_copy(idx_hbm.at[pl.ds(off, CHUNK)], idx_v.at[b])
        gathers[g] = pltpu.async_copy(table_hbm.at[idx_v.at[b]], rows_v.at[b], gsems[b])
        if g >= 1:
            pb = (g - 1) % NB
            gathers[g - 1].wait()
            writes[g - 1] = pltpu.async_copy(
                rows_v.at[pb], out_hbm.at[pl.ds(base + (g - 1) * CHUNK, CHUNK)], osems[pb]
            )
    last = NCHUNK - 1
    gathers[last].wait()
    writes[last] = pltpu.async_copy(
        rows_v.at[last % NB], out_hbm.at[pl.ds(base + last * CHUNK, CHUNK)], osems[last % NB]
    )
    for g in range(max(0, NCHUNK - NB), NCHUNK):
        writes[g].wait()


def kernel(hero_ids, table):
    tail = table[NBLK * 128 :, :].reshape(TAIL // 4, 128)
    lin128 = _relayout(table.T, tail)
    tab_lin = lin128.reshape(NUM_HEROES, EMBED_DIM)
    ids_hm = hero_ids.T.reshape(TOTAL).astype(jnp.int32)
    out_hm = _gather(ids_hm, tab_lin)
    return out_hm.reshape(HIST, BATCH, EMBED_DIM).transpose(1, 0, 2)


# speedup vs baseline: 3.2935x; 1.9807x over previous
import functools

import jax
import jax.numpy as jnp
from jax import lax
from jax.experimental import pallas as pl
from jax.experimental.pallas import tpu as pltpu
from jax.experimental.pallas import tpu_sc as plsc

NUM_HEROES = 1000000
EMBED_DIM = 32
BATCH = 16384
HIST = 20
TOTAL = BATCH * HIST

_info = plsc.get_sparse_core_info()
NC, NS = _info.num_cores, _info.num_subcores
NW = NC * NS

_mesh = plsc.VectorSubcoreMesh(core_axis_name="c", subcore_axis_name="s")

NBLK = NUM_HEROES // 128
TAIL = NUM_HEROES - NBLK * 128
BLK_PER_W = NBLK // NW
REM_BLK = NBLK - BLK_PER_W * NW


@functools.partial(
    pl.kernel,
    mesh=_mesh,
    out_type=jax.ShapeDtypeStruct((NUM_HEROES // 4, 128), jnp.float32),
    scratch_types=[
        [pltpu.VMEM((EMBED_DIM, 128), jnp.float32)] * 2,
        [pltpu.VMEM((EMBED_DIM, 128), jnp.float32)] * 2,
        [pltpu.SemaphoreType.DMA] * 2,
        [pltpu.SemaphoreType.DMA] * 2,
    ],
    compiler_params=pltpu.CompilerParams(
        use_tc_tiling_on_sc=True, needs_layout_passes=False
    ),
)
def _relayout(tabt_hbm, tail_hbm, lin_hbm, tvs, tbs, isems, osems):
    wid = lax.axis_index("s") * NC + lax.axis_index("c")
    first = wid * BLK_PER_W

    iota16 = lax.iota(jnp.int32, 16)
    def transpose(tv, tb):
        @plsc.parallel_loop(0, 256, unroll=8)
        def _(i):
            d = i & 15
            h0 = ((i >> 4) & 7) * 16
            s = (i >> 7) & 1
            perm = (iota16 + d) & 15
            hv = perm + h0
            rows = iota16 + 16 * s
            vals = plsc.load_gather(tv, [rows, hv])
            plsc.store_scatter(tb, [hv >> 2, (perm & 3) * 32 + rows], vals)

    def start_in(j, b):
        return pltpu.async_copy(
            tabt_hbm.at[:, pl.ds(j * 128, 128)], tvs[b], isems[b]
        )

    def start_out(j, b):
        return pltpu.async_copy(
            tbs[b], lin_hbm.at[pl.ds(j * 32, EMBED_DIM)], osems[b]
        )

    start_in(first, 0).wait()

    def body(i, carry):
        b0 = first + 2 * i
        cin1 = start_in(b0 + 1, 1)

        @pl.when(i > 0)
        def _():
            pltpu.make_async_copy(tabt_hbm.at[:, pl.ds(0, 128)], tvs[0], isems[0]).wait()
            pltpu.make_async_copy(tbs[0], lin_hbm.at[pl.ds(0, EMBED_DIM)], osems[0]).wait()

        transpose(tvs[0], tbs[0])
        start_out(b0, 0)
        nxt = jnp.minimum(b0 + 2, first + BLK_PER_W - 2)
        start_in(nxt, 0)
        cin1.wait()

        @pl.when(i > 0)
        def _():
            pltpu.make_async_copy(tbs[1], lin_hbm.at[pl.ds(0, EMBED_DIM)], osems[1]).wait()

        transpose(tvs[1], tbs[1])
        start_out(b0 + 1, 1)
        return carry

    lax.fori_loop(0, BLK_PER_W // 2, body, 0)
    pltpu.make_async_copy(tbs[0], lin_hbm.at[pl.ds(0, EMBED_DIM)], osems[0]).wait()
    pltpu.make_async_copy(tbs[1], lin_hbm.at[pl.ds(0, EMBED_DIM)], osems[1]).wait()
    pltpu.make_async_copy(tabt_hbm.at[:, pl.ds(0, 128)], tvs[0], isems[0]).wait()

    @pl.when(wid < REM_BLK)
    def _():
        j = NW * BLK_PER_W + wid
        pltpu.sync_copy(tabt_hbm.at[:, pl.ds(j * 128, 128)], tvs[0])
        transpose(tvs[0], tbs[0])
        pltpu.sync_copy(tbs[0], lin_hbm.at[pl.ds(j * 32, EMBED_DIM)])
    @pl.when(wid == REM_BLK)
    def _():
        pltpu.sync_copy(tail_hbm, lin_hbm.at[pl.ds(NBLK * 32, TAIL // 4), :])


PER_W = TOTAL // NW
CHUNK = 1280
NCHUNK = PER_W // CHUNK
NB = 2


@functools.partial(
    pl.kernel,
    mesh=_mesh,
    out_type=jax.ShapeDtypeStruct((TOTAL, EMBED_DIM), jnp.float32),
    scratch_types=[
        pltpu.VMEM((NB, CHUNK), jnp.int32),
        pltpu.VMEM((NB, CHUNK, EMBED_DIM), jnp.float32),
        [pltpu.SemaphoreType.DMA] * NB,
        [pltpu.SemaphoreType.DMA] * NB,
    ],
    compiler_params=pltpu.CompilerParams(use_tc_tiling_on_sc=False),
)
def _gather(idx_hbm, table_hbm, out_hbm, idx_v, rows_v, gsems, osems):
    wid = lax.axis_index("s") * NC + lax.axis_index("c")
    base = wid * PER_W

    gathers = [None] * NCHUNK
    writes = [None] * NCHUNK
    for g in range(NCHUNK):
        b = g % NB
        off = base + g * CHUNK
        if g >= NB:
            writes[g - NB].wait()
        pltpu.sync_copy(idx_hbm.at[pl.ds(off, CHUNK)], idx_v.at[b])
        gathers[g] = pltpu.async_copy(table_hbm.at[idx_v.at[b]], rows_v.at[b], gsems[b])
        if g >= 1:
            pb = (g - 1) % NB
            gathers[g - 1].wait()
            writes[g - 1] = pltpu.async_copy(
                rows_v.at[pb], out_hbm.at[pl.ds(base + (g - 1) * CHUNK, CHUNK)], osems[pb]
            )
    last = NCHUNK - 1
    gathers[last].wait()
    writes[last] = pltpu.async_copy(
        rows_v.at[last % NB], out_hbm.at[pl.ds(base + last * CHUNK, CHUNK)], osems[last % NB]
    )
    for g in range(max(0, NCHUNK - NB), NCHUNK):
        writes[g].wait()


def kernel(hero_ids, table):
    tail = table[NBLK * 128 :, :].reshape(TAIL // 4, 128)
    lin128 = _relayout(table.T, tail)
    tab_lin = lin128.reshape(NUM_HEROES, EMBED_DIM)
    ids_hm = hero_ids.T.reshape(TOTAL).astype(jnp.int32)
    out_hm = _gather(ids_hm, tab_lin)
    return out_hm.reshape(HIST, BATCH, EMBED_DIM).transpose(1, 0, 2)


# speedup vs baseline: 4.8292x; 1.4663x over previous
import functools

import jax
import jax.numpy as jnp
from jax import lax
from jax.experimental import pallas as pl
from jax.experimental.pallas import tpu as pltpu
from jax.experimental.pallas import tpu_sc as plsc

NUM_HEROES = 1000000
EMBED_DIM = 32
BATCH = 16384
HIST = 20
TOTAL = BATCH * HIST

_info = plsc.get_sparse_core_info()
NC, NS = _info.num_cores, _info.num_subcores
NW = NC * NS

_mesh = plsc.VectorSubcoreMesh(core_axis_name="c", subcore_axis_name="s")

NBLK = NUM_HEROES // 128
TAIL = NUM_HEROES - NBLK * 128
BLK_PER_W = NBLK // NW
REM_BLK = NBLK - BLK_PER_W * NW


@functools.partial(
    pl.kernel,
    mesh=_mesh,
    out_type=jax.ShapeDtypeStruct((NUM_HEROES // 4, 128), jnp.float32),
    scratch_types=[
        [pltpu.VMEM((EMBED_DIM, 128), jnp.float32)] * 2,
        [pltpu.VMEM((EMBED_DIM, 128), jnp.float32)] * 2,
        [pltpu.SemaphoreType.DMA] * 2,
        [pltpu.SemaphoreType.DMA] * 2,
    ],
    compiler_params=pltpu.CompilerParams(
        use_tc_tiling_on_sc=True, needs_layout_passes=False
    ),
)
def _relayout(tabt_hbm, tail_hbm, lin_hbm, tvs, tbs, isems, osems):
    wid = lax.axis_index("s") * NC + lax.axis_index("c")
    first = wid * BLK_PER_W

    iota16 = lax.iota(jnp.int32, 16)
    def transpose(tv, tb):
        @plsc.parallel_loop(0, 256, unroll=8)
        def _(i):
            d = i & 15
            h0 = ((i >> 4) & 7) * 16
            s = (i >> 7) & 1
            perm = (iota16 + d) & 15
            hv = perm + h0
            rows = iota16 + 16 * s
            vals = plsc.load_gather(tv, [rows, hv])
            plsc.store_scatter(tb, [hv >> 2, (perm & 3) * 32 + rows], vals)

    def start_in(j, b):
        return pltpu.async_copy(
            tabt_hbm.at[:, pl.ds(j * 128, 128)], tvs[b], isems[b]
        )

    def start_out(j, b):
        return pltpu.async_copy(
            tbs[b], lin_hbm.at[pl.ds(j * 32, EMBED_DIM)], osems[b]
        )

    start_in(first, 0).wait()

    def body(i, carry):
        b0 = first + 2 * i
        cin1 = start_in(b0 + 1, 1)

        @pl.when(i > 0)
        def _():
            pltpu.make_async_copy(tabt_hbm.at[:, pl.ds(0, 128)], tvs[0], isems[0]).wait()
            pltpu.make_async_copy(tbs[0], lin_hbm.at[pl.ds(0, EMBED_DIM)], osems[0]).wait()

        transpose(tvs[0], tbs[0])
        start_out(b0, 0)
        nxt = jnp.minimum(b0 + 2, first + BLK_PER_W - 2)
        start_in(nxt, 0)
        cin1.wait()

        @pl.when(i > 0)
        def _():
            pltpu.make_async_copy(tbs[1], lin_hbm.at[pl.ds(0, EMBED_DIM)], osems[1]).wait()

        transpose(tvs[1], tbs[1])
        start_out(b0 + 1, 1)
        return carry

    lax.fori_loop(0, BLK_PER_W // 2, body, 0)
    pltpu.make_async_copy(tbs[0], lin_hbm.at[pl.ds(0, EMBED_DIM)], osems[0]).wait()
    pltpu.make_async_copy(tbs[1], lin_hbm.at[pl.ds(0, EMBED_DIM)], osems[1]).wait()
    pltpu.make_async_copy(tabt_hbm.at[:, pl.ds(0, 128)], tvs[0], isems[0]).wait()

    @pl.when(wid < REM_BLK)
    def _():
        j = NW * BLK_PER_W + wid
        pltpu.sync_copy(tabt_hbm.at[:, pl.ds(j * 128, 128)], tvs[0])
        transpose(tvs[0], tbs[0])
        pltpu.sync_copy(tbs[0], lin_hbm.at[pl.ds(j * 32, EMBED_DIM)])
    @pl.when(wid == REM_BLK)
    def _():
        pltpu.sync_copy(tail_hbm, lin_hbm.at[pl.ds(NBLK * 32, TAIL // 4), :])


PER_W = TOTAL // NW
CHUNK = 1280
NCHUNK = PER_W // CHUNK
NB = 2
BPC = CHUNK // 128

OUT5 = (HIST, EMBED_DIM // 8, BATCH // 128, 8, 128)


@functools.partial(
    pl.kernel,
    mesh=_mesh,
    out_type=jax.ShapeDtypeStruct(OUT5, jnp.float32),
    scratch_types=[
        pltpu.VMEM((NB, CHUNK), jnp.int32),
        pltpu.VMEM((NB, CHUNK, EMBED_DIM), jnp.float32),
        [pltpu.VMEM((EMBED_DIM, 128), jnp.float32)] * 2,
        [pltpu.SemaphoreType.DMA] * NB,
        [pltpu.SemaphoreType.DMA] * 2,
    ],
    compiler_params=pltpu.CompilerParams(
        use_tc_tiling_on_sc=False, needs_layout_passes=False
    ),
)
def _gather(idx_hbm, table_hbm, out_hbm, idx_v, rows_v, tbos, gsems, osems):
    wid = lax.axis_index("s") * NC + lax.axis_index("c")
    base = wid * PER_W
    iota16 = lax.iota(jnp.int32, 16)

    def process_chunk(c, pb, is_first):
        rows_ref = rows_v.at[pb]

        def drain(u):
            for ct in range(4):
                pltpu.make_async_copy(
                    tbos[u].at[pl.ds(ct * 8, 8), :], out_hbm.at[0, ct, 0], osems[u]
                ).wait()

        def pair(i, carry):
            for u in range(2):
                k = 2 * i + u
                if is_first:
                    @pl.when(i > 0)
                    def _():
                        drain(u)
                else:
                    drain(u)
                koff = k * 128

                @plsc.parallel_loop(0, 256, unroll=8)
                def _(i2):
                    d = i2 & 15
                    b0 = ((i2 >> 4) & 7) * 16
                    s = (i2 >> 7) & 1
                    perm = (iota16 + d) & 15
                    bv = b0 + perm
                    cv = iota16 + 16 * s
                    vals = plsc.load_gather(rows_ref, [koff + bv, cv])
                    plsc.store_scatter(tbos[u], [cv, bv], vals)

                p_blk = base + c * CHUNK + koff
                h = p_blk >> 14
                bb = (p_blk & (BATCH - 1)) >> 7
                for ct in range(4):
                    pltpu.async_copy(
                        tbos[u].at[pl.ds(ct * 8, 8), :],
                        out_hbm.at[h, ct, bb],
                        osems[u],
                    )
            return carry

        lax.fori_loop(0, BPC // 2, pair, 0)

    gathers = [None] * NCHUNK
    for g in range(NCHUNK):
        b = g % NB
        off = base + g * CHUNK
        pltpu.sync_copy(idx_hbm.at[pl.ds(off, CHUNK)], idx_v.at[b])
        gathers[g] = pltpu.async_copy(table_hbm.at[idx_v.at[b]], rows_v.at[b], gsems[b])
        if g >= 1:
            gathers[g - 1].wait()
            process_chunk(g - 1, (g - 1) % NB, g == 1)
    gathers[NCHUNK - 1].wait()
    process_chunk(NCHUNK - 1, (NCHUNK - 1) % NB, False)
    for u in range(2):
        for ct in range(4):
            pltpu.make_async_copy(
                tbos[u].at[pl.ds(ct * 8, 8), :], out_hbm.at[0, ct, 0], osems[u]
            ).wait()


def kernel(hero_ids, table):
    tail = table[NBLK * 128 :, :].reshape(TAIL // 4, 128)
    lin128 = _relayout(table.T, tail)
    tab_lin = lin128.reshape(NUM_HEROES, EMBED_DIM)
    ids_hm = hero_ids.T.reshape(TOTAL).astype(jnp.int32)
    out5 = _gather(ids_hm, tab_lin)
    return out5.transpose(2, 4, 0, 1, 3).reshape(BATCH, HIST, EMBED_DIM)


# speedup vs baseline: 5.5848x; 1.1565x over previous
import functools

import jax
import jax.numpy as jnp
from jax import lax
from jax.experimental import pallas as pl
from jax.experimental.pallas import tpu as pltpu
from jax.experimental.pallas import tpu_sc as plsc

NUM_HEROES = 1000000
EMBED_DIM = 32
BATCH = 16384
HIST = 20
TOTAL = BATCH * HIST

_info = plsc.get_sparse_core_info()
NC, NS = _info.num_cores, _info.num_subcores
NW = NC * NS

_mesh = plsc.VectorSubcoreMesh(core_axis_name="c", subcore_axis_name="s")

NBLK = NUM_HEROES // 128
TAIL = NUM_HEROES - NBLK * 128
BLK_PER_W = NBLK // NW
REM_BLK = NBLK - BLK_PER_W * NW


@functools.partial(
    pl.kernel,
    mesh=_mesh,
    out_type=jax.ShapeDtypeStruct((NUM_HEROES // 4, 128), jnp.float32),
    scratch_types=[
        [pltpu.VMEM((EMBED_DIM, 128), jnp.float32)] * 2,
        [pltpu.VMEM((EMBED_DIM, 128), jnp.float32)] * 2,
        [pltpu.SemaphoreType.DMA] * 2,
        [pltpu.SemaphoreType.DMA] * 2,
    ],
    compiler_params=pltpu.CompilerParams(
        use_tc_tiling_on_sc=True, needs_layout_passes=False
    ),
)
def _relayout(tabt_hbm, tail_hbm, lin_hbm, tvs, tbs, isems, osems):
    wid = lax.axis_index("s") * NC + lax.axis_index("c")
    first = wid * BLK_PER_W

    iota16 = lax.iota(jnp.int32, 16)
    def transpose(tv, tb):
        @plsc.parallel_loop(0, 256, unroll=8)
        def _(i):
            d = (i >> 3) & 15
            h0 = (i & 7) * 16
            s = (i >> 7) & 1
            perm = (iota16 + d) & 15
            hv = perm + h0
            rows = iota16 + 16 * s
            vals = plsc.load_gather(tv, [rows, hv])
            plsc.store_scatter(tb, [hv >> 2, (perm & 3) * 32 + rows], vals)

    def start_in(j, b):
        return pltpu.async_copy(
            tabt_hbm.at[:, pl.ds(j * 128, 128)], tvs[b], isems[b]
        )

    def start_out(j, b):
        return pltpu.async_copy(
            tbs[b], lin_hbm.at[pl.ds(j * 32, EMBED_DIM)], osems[b]
        )

    start_in(first, 0).wait()

    def body(i, carry):
        b0 = first + 2 * i
        cin1 = start_in(b0 + 1, 1)

        @pl.when(i > 0)
        def _():
            pltpu.make_async_copy(tabt_hbm.at[:, pl.ds(0, 128)], tvs[0], isems[0]).wait()
            pltpu.make_async_copy(tbs[0], lin_hbm.at[pl.ds(0, EMBED_DIM)], osems[0]).wait()

        transpose(tvs[0], tbs[0])
        start_out(b0, 0)
        nxt = jnp.minimum(b0 + 2, first + BLK_PER_W - 2)
        start_in(nxt, 0)
        cin1.wait()

        @pl.when(i > 0)
        def _():
            pltpu.make_async_copy(tbs[1], lin_hbm.at[pl.ds(0, EMBED_DIM)], osems[1]).wait()

        transpose(tvs[1], tbs[1])
        start_out(b0 + 1, 1)
        return carry

    lax.fori_loop(0, BLK_PER_W // 2, body, 0)
    pltpu.make_async_copy(tbs[0], lin_hbm.at[pl.ds(0, EMBED_DIM)], osems[0]).wait()
    pltpu.make_async_copy(tbs[1], lin_hbm.at[pl.ds(0, EMBED_DIM)], osems[1]).wait()
    pltpu.make_async_copy(tabt_hbm.at[:, pl.ds(0, 128)], tvs[0], isems[0]).wait()

    @pl.when(wid < REM_BLK)
    def _():
        j = NW * BLK_PER_W + wid
        pltpu.sync_copy(tabt_hbm.at[:, pl.ds(j * 128, 128)], tvs[0])
        transpose(tvs[0], tbs[0])
        pltpu.sync_copy(tbs[0], lin_hbm.at[pl.ds(j * 32, EMBED_DIM)])
    @pl.when(wid == REM_BLK)
    def _():
        pltpu.sync_copy(tail_hbm, lin_hbm.at[pl.ds(NBLK * 32, TAIL // 4), :])


PER_W = TOTAL // NW
CHUNK = 1280
NCHUNK = PER_W // CHUNK
NB = 2
BPC = CHUNK // 128

OUT5 = (HIST, EMBED_DIM // 8, BATCH // 128, 8, 128)


@functools.partial(
    pl.kernel,
    mesh=_mesh,
    out_type=jax.ShapeDtypeStruct(OUT5, jnp.float32),
    scratch_types=[
        pltpu.VMEM((NB, CHUNK), jnp.int32),
        pltpu.VMEM((NB, CHUNK, EMBED_DIM), jnp.float32),
        [pltpu.VMEM((EMBED_DIM, 128), jnp.float32)] * 2,
        [pltpu.SemaphoreType.DMA] * NB,
        [pltpu.SemaphoreType.DMA] * 2,
    ],
    compiler_params=pltpu.CompilerParams(
        use_tc_tiling_on_sc=False, needs_layout_passes=False
    ),
)
def _gather(idx_hbm, table_hbm, out_hbm, idx_v, rows_v, tbos, gsems, osems):
    wid = lax.axis_index("s") * NC + lax.axis_index("c")
    base = wid * PER_W
    iota16 = lax.iota(jnp.int32, 16)

    def process_chunk(c, pb, is_first):
        rows_ref = rows_v.at[pb]

        def drain(u):
            for ct in range(4):
                pltpu.make_async_copy(
                    tbos[u].at[pl.ds(ct * 8, 8), :], out_hbm.at[0, ct, 0], osems[u]
                ).wait()

        def pair(i, carry):
            for u in range(2):
                k = 2 * i + u
                if is_first:
                    @pl.when(i > 0)
                    def _():
                        drain(u)
                else:
                    drain(u)
                koff = k * 128

                @plsc.parallel_loop(0, 256, unroll=8)
                def _(i2):
                    d = (i2 >> 3) & 15
                    b0 = (i2 & 7) * 16
                    s = (i2 >> 7) & 1
                    perm = (iota16 + d) & 15
                    bv = b0 + perm
                    cv = iota16 + 16 * s
                    vals = plsc.load_gather(rows_ref, [koff + bv, cv])
                    plsc.store_scatter(tbos[u], [cv, bv], vals)

                p_blk = base + c * CHUNK + koff
                h = p_blk >> 14
                bb = (p_blk & (BATCH - 1)) >> 7
                for ct in range(4):
                    pltpu.async_copy(
                        tbos[u].at[pl.ds(ct * 8, 8), :],
                        out_hbm.at[h, ct, bb],
                        osems[u],
                    )
            return carry

        lax.fori_loop(0, BPC // 2, pair, 0)

    gathers = [None] * NCHUNK
    for g in range(NCHUNK):
        b = g % NB
        off = base + g * CHUNK
        pltpu.sync_copy(idx_hbm.at[pl.ds(off, CHUNK)], idx_v.at[b])
        gathers[g] = pltpu.async_copy(table_hbm.at[idx_v.at[b]], rows_v.at[b], gsems[b])
        if g >= 1:
            gathers[g - 1].wait()
            process_chunk(g - 1, (g - 1) % NB, g == 1)
    gathers[NCHUNK - 1].wait()
    process_chunk(NCHUNK - 1, (NCHUNK - 1) % NB, False)
    for u in range(2):
        for ct in range(4):
            pltpu.make_async_copy(
                tbos[u].at[pl.ds(ct * 8, 8), :], out_hbm.at[0, ct, 0], osems[u]
            ).wait()


def kernel(hero_ids, table):
    tail = table[NBLK * 128 :, :].reshape(TAIL // 4, 128)
    lin128 = _relayout(table.T, tail)
    tab_lin = lin128.reshape(NUM_HEROES, EMBED_DIM)
    ids_hm = hero_ids.T.reshape(TOTAL).astype(jnp.int32)
    out5 = _gather(ids_hm, tab_lin)
    return out5.transpose(2, 4, 0, 1, 3).reshape(BATCH, HIST, EMBED_DIM)


# speedup vs baseline: 7.2507x; 1.2983x over previous
import functools

import jax
import jax.numpy as jnp
from jax import lax
from jax.experimental import pallas as pl
from jax.experimental.pallas import tpu as pltpu
from jax.experimental.pallas import tpu_sc as plsc

NUM_HEROES = 1000000
EMBED_DIM = 32
BATCH = 16384
HIST = 20
TOTAL = BATCH * HIST

_info = plsc.get_sparse_core_info()
NC, NS = _info.num_cores, _info.num_subcores
NW = NC * NS

_mesh = plsc.VectorSubcoreMesh(core_axis_name="c", subcore_axis_name="s")

NBLK = NUM_HEROES // 128
TAIL = NUM_HEROES - NBLK * 128
GRP = 4
NGRP = NBLK // GRP
GRP_PER_W = NGRP // NW
REM_GRP = NGRP - GRP_PER_W * NW


@functools.partial(
    pl.kernel,
    mesh=_mesh,
    out_type=jax.ShapeDtypeStruct((NUM_HEROES // 4, 128), jnp.float32),
    scratch_types=[
        [pltpu.VMEM((EMBED_DIM, GRP * 128), jnp.float32)] * 2,
        [pltpu.VMEM((GRP * 32, 128), jnp.float32)] * 2,
        [pltpu.SemaphoreType.DMA] * 2,
        [pltpu.SemaphoreType.DMA] * 2,
    ],
    compiler_params=pltpu.CompilerParams(
        use_tc_tiling_on_sc=True, needs_layout_passes=False
    ),
)
def _relayout(tabt_hbm, tail_hbm, lin_hbm, tvs, tbs, isems, osems):
    wid = lax.axis_index("s") * NC + lax.axis_index("c")
    first = wid * GRP_PER_W

    iota16 = lax.iota(jnp.int32, 16)

    def transpose(tv, tb):
        @plsc.parallel_loop(0, GRP * 256, unroll=8)
        def _(i):
            d = (i >> 3) & 15
            h0 = (i & 7) * 16
            s = (i >> 7) & 1
            jj = (i >> 8) & (GRP - 1)
            perm = (iota16 + d) & 15
            hv = perm + h0
            rows = iota16 + 16 * s
            vals = plsc.load_gather(tv, [rows, jj * 128 + hv])
            plsc.store_scatter(
                tb, [jj * 32 + (hv >> 2), (perm & 3) * 32 + rows], vals
            )

    def start_in(j, b):
        return pltpu.async_copy(
            tabt_hbm.at[:, pl.ds(j * (GRP * 128), GRP * 128)], tvs[b], isems[b]
        )

    def start_out(j, b):
        return pltpu.async_copy(
            tbs[b], lin_hbm.at[pl.ds(j * (GRP * 32), GRP * 32)], osems[b]
        )

    def drain_out(b):
        pltpu.make_async_copy(tbs[b], lin_hbm.at[pl.ds(0, GRP * 32)], osems[b]).wait()

    def drain_in(b):
        pltpu.make_async_copy(
            tabt_hbm.at[:, pl.ds(0, GRP * 128)], tvs[b], isems[b]
        ).wait()

    start_in(first, 0).wait()

    def body(i, carry):
        g0 = first + 2 * i
        cin1 = start_in(g0 + 1, 1)

        @pl.when(i > 0)
        def _():
            drain_in(0)
            drain_out(0)

        transpose(tvs[0], tbs[0])
        start_out(g0, 0)
        start_in(g0 + 2, 0)
        cin1.wait()

        @pl.when(i > 0)
        def _():
            drain_out(1)

        transpose(tvs[1], tbs[1])
        start_out(g0 + 1, 1)
        return carry

    lax.fori_loop(0, GRP_PER_W // 2, body, 0)
    drain_out(0)
    drain_out(1)
    drain_in(0)
    transpose(tvs[0], tbs[0])
    pltpu.sync_copy(
        tbs[0], lin_hbm.at[pl.ds((first + GRP_PER_W - 1) * (GRP * 32), GRP * 32)]
    )

    @pl.when(wid < REM_GRP)
    def _():
        j = NW * GRP_PER_W + wid
        pltpu.sync_copy(tabt_hbm.at[:, pl.ds(j * (GRP * 128), GRP * 128)], tvs[1])
        transpose(tvs[1], tbs[1])
        pltpu.sync_copy(tbs[1], lin_hbm.at[pl.ds(j * (GRP * 32), GRP * 32)])
    @pl.when(wid == REM_GRP)
    def _():
        pltpu.sync_copy(tail_hbm, lin_hbm.at[pl.ds(NBLK * 32, TAIL // 4), :])


PER_W = TOTAL // NW
CHUNK = 1280
NCHUNK = PER_W // CHUNK
NB = 2
BPC = CHUNK // 128

OUT5 = (HIST, EMBED_DIM // 8, BATCH // 128, 8, 128)


@functools.partial(
    pl.kernel,
    mesh=_mesh,
    out_type=jax.ShapeDtypeStruct(OUT5, jnp.float32),
    scratch_types=[
        pltpu.VMEM((NB, CHUNK), jnp.int32),
        pltpu.VMEM((NB, CHUNK, EMBED_DIM), jnp.float32),
        [pltpu.VMEM((EMBED_DIM, 128), jnp.float32)] * 2,
        [pltpu.SemaphoreType.DMA] * NB,
        [pltpu.SemaphoreType.DMA] * 2,
    ],
    compiler_params=pltpu.CompilerParams(
        use_tc_tiling_on_sc=False, needs_layout_passes=False
    ),
)
def _gather(idx_hbm, table_hbm, out_hbm, idx_v, rows_v, tbos, gsems, osems):
    wid = lax.axis_index("s") * NC + lax.axis_index("c")
    base = wid * PER_W
    iota16 = lax.iota(jnp.int32, 16)

    def process_chunk(c, pb, is_first):
        rows_ref = rows_v.at[pb]

        def drain(u):
            for ct in range(4):
                pltpu.make_async_copy(
                    tbos[u].at[pl.ds(ct * 8, 8), :], out_hbm.at[0, ct, 0], osems[u]
                ).wait()

        def pair(i, carry):
            for u in range(2):
                k = 2 * i + u
                if is_first:
                    @pl.when(i > 0)
                    def _():
                        drain(u)
                else:
                    drain(u)
                koff = k * 128

                @plsc.parallel_loop(0, 256, unroll=8)
                def _(i2):
                    d = (i2 >> 3) & 15
                    b0 = (i2 & 7) * 16
                    s = (i2 >> 7) & 1
                    perm = (iota16 + d) & 15
                    bv = b0 + perm
                    cv = iota16 + 16 * s
                    vals = plsc.load_gather(rows_ref, [koff + bv, cv])
                    plsc.store_scatter(tbos[u], [cv, bv], vals)

                p_blk = base + c * CHUNK + koff
                h = p_blk >> 14
                bb = (p_blk & (BATCH - 1)) >> 7
                for ct in range(4):
                    pltpu.async_copy(
                        tbos[u].at[pl.ds(ct * 8, 8), :],
                        out_hbm.at[h, ct, bb],
                        osems[u],
                    )
            return carry

        lax.fori_loop(0, BPC // 2, pair, 0)

    gathers = [None] * NCHUNK
    for g in range(NCHUNK):
        b = g % NB
        off = base + g * CHUNK
        pltpu.sync_copy(idx_hbm.at[pl.ds(off, CHUNK)], idx_v.at[b])
        gathers[g] = pltpu.async_copy(table_hbm.at[idx_v.at[b]], rows_v.at[b], gsems[b])
        if g >= 1:
            gathers[g - 1].wait()
            process_chunk(g - 1, (g - 1) % NB, g == 1)
    gathers[NCHUNK - 1].wait()
    process_chunk(NCHUNK - 1, (NCHUNK - 1) % NB, False)
    for u in range(2):
        for ct in range(4):
            pltpu.make_async_copy(
                tbos[u].at[pl.ds(ct * 8, 8), :], out_hbm.at[0, ct, 0], osems[u]
            ).wait()


def kernel(hero_ids, table):
    tail = table[NBLK * 128 :, :].reshape(TAIL // 4, 128)
    lin128 = _relayout(table.T, tail)
    tab_lin = lin128.reshape(NUM_HEROES, EMBED_DIM)
    ids_hm = hero_ids.T.reshape(TOTAL).astype(jnp.int32)
    out5 = _gather(ids_hm, tab_lin)
    return out5.transpose(2, 4, 0, 1, 3).reshape(BATCH, HIST, EMBED_DIM)


# speedup vs baseline: 7.3530x; 1.0141x over previous
import functools

import jax
import jax.numpy as jnp
from jax import lax
from jax.experimental import pallas as pl
from jax.experimental.pallas import tpu as pltpu
from jax.experimental.pallas import tpu_sc as plsc

NUM_HEROES = 1000000
EMBED_DIM = 32
BATCH = 16384
HIST = 20
TOTAL = BATCH * HIST

_info = plsc.get_sparse_core_info()
NC, NS = _info.num_cores, _info.num_subcores
NW = NC * NS

_mesh = plsc.VectorSubcoreMesh(core_axis_name="c", subcore_axis_name="s")

NBLK = NUM_HEROES // 128
TAIL = NUM_HEROES - NBLK * 128
GRP = 4
NGRP = NBLK // GRP
GRP_PER_W = NGRP // NW
REM_GRP = NGRP - GRP_PER_W * NW


@functools.partial(
    pl.kernel,
    mesh=_mesh,
    out_type=jax.ShapeDtypeStruct((NUM_HEROES // 4, 128), jnp.float32),
    scratch_types=[
        [pltpu.VMEM((EMBED_DIM, GRP * 128), jnp.float32)] * 2,
        [pltpu.VMEM((GRP * 32, 128), jnp.float32)] * 2,
        [pltpu.SemaphoreType.DMA] * 2,
        [pltpu.SemaphoreType.DMA] * 2,
    ],
    compiler_params=pltpu.CompilerParams(
        use_tc_tiling_on_sc=True, needs_layout_passes=False
    ),
)
def _relayout(tabt_hbm, tail_hbm, lin_hbm, tvs, tbs, isems, osems):
    wid = lax.axis_index("s") * NC + lax.axis_index("c")
    first = wid * GRP_PER_W

    iota16 = lax.iota(jnp.int32, 16)

    def transpose(tv, tb):
        @plsc.parallel_loop(0, GRP * 256, unroll=16)
        def _(i):
            d = (i >> 3) & 15
            h0 = (i & 7) * 16
            s = (i >> 7) & 1
            jj = (i >> 8) & (GRP - 1)
            perm = (iota16 + d) & 15
            hv = perm + h0
            rows = iota16 + 16 * s
            vals = plsc.load_gather(tv, [rows, jj * 128 + hv])
            plsc.store_scatter(
                tb, [jj * 32 + (hv >> 2), (perm & 3) * 32 + rows], vals
            )

    def start_in(j, b):
        return pltpu.async_copy(
            tabt_hbm.at[:, pl.ds(j * (GRP * 128), GRP * 128)], tvs[b], isems[b]
        )

    def start_out(j, b):
        return pltpu.async_copy(
            tbs[b], lin_hbm.at[pl.ds(j * (GRP * 32), GRP * 32)], osems[b]
        )

    def drain_out(b):
        pltpu.make_async_copy(tbs[b], lin_hbm.at[pl.ds(0, GRP * 32)], osems[b]).wait()

    def drain_in(b):
        pltpu.make_async_copy(
            tabt_hbm.at[:, pl.ds(0, GRP * 128)], tvs[b], isems[b]
        ).wait()

    start_in(first, 0).wait()

    def body(i, carry):
        g0 = first + 2 * i
        cin1 = start_in(g0 + 1, 1)

        @pl.when(i > 0)
        def _():
            drain_in(0)
            drain_out(0)

        transpose(tvs[0], tbs[0])
        start_out(g0, 0)
        start_in(g0 + 2, 0)
        cin1.wait()

        @pl.when(i > 0)
        def _():
            drain_out(1)

        transpose(tvs[1], tbs[1])
        start_out(g0 + 1, 1)
        return carry

    lax.fori_loop(0, GRP_PER_W // 2, body, 0)
    drain_out(0)
    drain_out(1)
    drain_in(0)
    transpose(tvs[0], tbs[0])
    pltpu.sync_copy(
        tbs[0], lin_hbm.at[pl.ds((first + GRP_PER_W - 1) * (GRP * 32), GRP * 32)]
    )

    @pl.when(wid < REM_GRP)
    def _():
        j = NW * GRP_PER_W + wid
        pltpu.sync_copy(tabt_hbm.at[:, pl.ds(j * (GRP * 128), GRP * 128)], tvs[1])
        transpose(tvs[1], tbs[1])
        pltpu.sync_copy(tbs[1], lin_hbm.at[pl.ds(j * (GRP * 32), GRP * 32)])
    @pl.when(wid == REM_GRP)
    def _():
        pltpu.sync_copy(tail_hbm, lin_hbm.at[pl.ds(NBLK * 32, TAIL // 4), :])


PER_W = TOTAL // NW
CHUNK = 1280
NCHUNK = PER_W // CHUNK
NB = 2
BPC = CHUNK // 128

OUT5 = (HIST, EMBED_DIM // 8, BATCH // 128, 8, 128)


@functools.partial(
    pl.kernel,
    mesh=_mesh,
    out_type=jax.ShapeDtypeStruct(OUT5, jnp.float32),
    scratch_types=[
        pltpu.VMEM((NB, CHUNK), jnp.int32),
        pltpu.VMEM((NB, CHUNK, EMBED_DIM), jnp.float32),
        [pltpu.VMEM((EMBED_DIM, 128), jnp.float32)] * 2,
        [pltpu.SemaphoreType.DMA] * NB,
        [pltpu.SemaphoreType.DMA] * 2,
    ],
    compiler_params=pltpu.CompilerParams(
        use_tc_tiling_on_sc=False, needs_layout_passes=False
    ),
)
def _gather(idx_hbm, table_hbm, out_hbm, idx_v, rows_v, tbos, gsems, osems):
    wid = lax.axis_index("s") * NC + lax.axis_index("c")
    base = wid * PER_W
    iota16 = lax.iota(jnp.int32, 16)

    def process_chunk(c, pb, is_first):
        rows_ref = rows_v.at[pb]

        def drain(u):
            for ct in range(4):
                pltpu.make_async_copy(
                    tbos[u].at[pl.ds(ct * 8, 8), :], out_hbm.at[0, ct, 0], osems[u]
                ).wait()

        def pair(i, carry):
            for u in range(2):
                k = 2 * i + u
                if is_first:
                    @pl.when(i > 0)
                    def _():
                        drain(u)
                else:
                    drain(u)
                koff = k * 128

                @plsc.parallel_loop(0, 256, unroll=16)
                def _(i2):
                    d = (i2 >> 3) & 15
                    b0 = (i2 & 7) * 16
                    s = (i2 >> 7) & 1
                    perm = (iota16 + d) & 15
                    bv = b0 + perm
                    cv = iota16 + 16 * s
                    vals = plsc.load_gather(rows_ref, [koff + bv, cv])
                    plsc.store_scatter(tbos[u], [cv, bv], vals)

                p_blk = base + c * CHUNK + koff
                h = p_blk >> 14
                bb = (p_blk & (BATCH - 1)) >> 7
                for ct in range(4):
                    pltpu.async_copy(
                        tbos[u].at[pl.ds(ct * 8, 8), :],
                        out_hbm.at[h, ct, bb],
                        osems[u],
                    )
            return carry

        lax.fori_loop(0, BPC // 2, pair, 0)

    gathers = [None] * NCHUNK
    for g in range(NCHUNK):
        b = g % NB
        off = base + g * CHUNK
        pltpu.sync_copy(idx_hbm.at[pl.ds(off, CHUNK)], idx_v.at[b])
        gathers[g] = pltpu.async_copy(table_hbm.at[idx_v.at[b]], rows_v.at[b], gsems[b])
        if g >= 1:
            gathers[g - 1].wait()
            process_chunk(g - 1, (g - 1) % NB, g == 1)
    gathers[NCHUNK - 1].wait()
    process_chunk(NCHUNK - 1, (NCHUNK - 1) % NB, False)
    for u in range(2):
        for ct in range(4):
            pltpu.make_async_copy(
                tbos[u].at[pl.ds(ct * 8, 8), :], out_hbm.at[0, ct, 0], osems[u]
            ).wait()


def kernel(hero_ids, table):
    tail = table[NBLK * 128 :, :].reshape(TAIL // 4, 128)
    lin128 = _relayout(table.T, tail)
    tab_lin = lin128.reshape(NUM_HEROES, EMBED_DIM)
    ids_hm = hero_ids.T.reshape(TOTAL).astype(jnp.int32)
    out5 = _gather(ids_hm, tab_lin)
    return out5.transpose(2, 4, 0, 1, 3).reshape(BATCH, HIST, EMBED_DIM)
